# baseline (speedup 1.0000x reference)
import functools

import jax
import jax.numpy as jnp
from jax import lax
from jax.experimental import pallas as pl
from jax.experimental.pallas import tpu as pltpu
from jax.experimental.pallas import tpu_sc as plsc

_B = 4
_S = 2048
_D = 768
_N = _B * _S
_L = 16
_NC = 2
_NS = 16
_NW = _NC * _NS
_PER_W = _N // _NW
_C = 16
_NCH = _PER_W // _C
_CVECS = _D // _L


def _make_sc_embed():
    mesh = plsc.VectorSubcoreMesh(core_axis_name="c", subcore_axis_name="s")

    @functools.partial(
        pl.kernel,
        mesh=mesh,
        out_type=jax.ShapeDtypeStruct((_N, _D), jnp.float32),
        scratch_types=[
            pltpu.VMEM((_NCH, _C), jnp.int32),
            pltpu.VMEM((_NCH, _C), jnp.int32),
            pltpu.VMEM((_C, _D), jnp.float32),
            pltpu.VMEM((_C, _D), jnp.float32),
            pltpu.VMEM((_C, _D), jnp.float32),
            pltpu.VMEM((_C, _D), jnp.float32),
            pltpu.VMEM((_C, _D), jnp.float32),
            pltpu.VMEM((_C, _D), jnp.float32),
            pltpu.SemaphoreType.DMA,
            pltpu.SemaphoreType.DMA,
            pltpu.SemaphoreType.DMA,
            pltpu.SemaphoreType.DMA,
            pltpu.SemaphoreType.DMA,
            pltpu.SemaphoreType.DMA,
            pltpu.SemaphoreType.DMA,
            pltpu.SemaphoreType.DMA,
        ],
    )
    def sc_embed(ids_hbm, sids_hbm, tok_hbm, seg_hbm, pos_hbm, out_hbm,
                 idx_v, sidx_v, tok0, tok1, seg0, seg1, pos0, pos1,
                 st0, st1, ss0, ss1, sp0, sp1, so0, so1):
        wid = lax.axis_index("s") * _NC + lax.axis_index("c")
        base = wid * _PER_W
        s0 = base % _S

        toks = (tok0, tok1)
        segs = (seg0, seg1)
        poss = (pos0, pos1)
        sts = (st0, st1)
        sss = (ss0, ss1)
        sps = (sp0, sp1)
        sos = (so0, so1)

        pltpu.sync_copy(ids_hbm.at[wid], idx_v)
        pltpu.sync_copy(sids_hbm.at[wid], sidx_v)

        def gathers(j, b):
            return (
                pltpu.make_async_copy(
                    tok_hbm.at[idx_v.at[j]], toks[b], sts[b]),
                pltpu.make_async_copy(
                    seg_hbm.at[sidx_v.at[j]], segs[b], sss[b]),
                pltpu.make_async_copy(
                    pos_hbm.at[pl.ds(s0 + j * _C, _C)], poss[b], sps[b]),
            )

        def out_copy(j, b):
            return pltpu.make_async_copy(
                toks[b], out_hbm.at[pl.ds(base + j * _C, _C)], sos[b])

        for cp in gathers(0, 0):
            cp.start()

        def outer(i, carry):
            for b in (0, 1):
                j = i * 2 + b
                nb = 1 - b
                for cp in gathers(j, b):
                    cp.wait()

                @pl.when(j >= 1)
                def _wait_prev_out():
                    out_copy(j - 1, nb).wait()

                @pl.when(j + 1 < _NCH)
                def _issue_next():
                    for cp in gathers(j + 1, nb):
                        cp.start()

                tok_b, seg_b, pos_b = toks[b], segs[b], poss[b]

                def row_body(r, carry2):
                    for cb in range(_CVECS):
                        sl = pl.ds(cb * _L, _L)
                        tok_b[r, sl] = tok_b[r, sl] + seg_b[r, sl] + pos_b[r, sl]
                    return carry2

                lax.fori_loop(0, _C, row_body, None)
                out_copy(j, b).start()
            return carry

        lax.fori_loop(0, _NCH // 2, outer, None)
        out_copy(_NCH - 1, (_NCH - 1) % 2).wait()

    return sc_embed


_sc_embed = _make_sc_embed()


@jax.jit
def kernel(input_ids, segment_ids, token_table, segment_table,
           position_table):
    ids = input_ids.reshape(_NW, _NCH, _C).astype(jnp.int32)
    sids = segment_ids.reshape(_NW, _NCH, _C).astype(jnp.int32)
    out = _sc_embed(ids, sids, token_table, segment_table, position_table)
    return out.reshape(_B, _S, _D)


# speedup vs baseline: 4.0940x; 4.0940x over previous
import functools

import jax
import jax.numpy as jnp
from jax import lax
from jax.experimental import pallas as pl
from jax.experimental.pallas import tpu as pltpu
from jax.experimental.pallas import tpu_sc as plsc

_B = 4
_S = 2048
_D = 768
_N = _B * _S
_L = 16
_NC = 2
_NS = 16
_NW = _NC * _NS
_PER_W = _N // _NW
_C = 16
_NCH = _PER_W // _C
_CVECS = _D // _L


def _make_sc_embed():
    mesh = plsc.VectorSubcoreMesh(core_axis_name="c", subcore_axis_name="s")

    @functools.partial(
        pl.kernel,
        mesh=mesh,
        out_type=jax.ShapeDtypeStruct((_N, _D), jnp.float32),
        scratch_types=[
            pltpu.VMEM((_NCH, _C), jnp.int32),
            pltpu.VMEM((_NCH, _C), jnp.int32),
            pltpu.VMEM((_C, _D), jnp.float32),
            pltpu.VMEM((_C, _D), jnp.float32),
            pltpu.VMEM((_C, _D), jnp.float32),
            pltpu.VMEM((_C, _D), jnp.float32),
            pltpu.VMEM((2, _D), jnp.float32),
            pltpu.SemaphoreType.DMA,
            pltpu.SemaphoreType.DMA,
            pltpu.SemaphoreType.DMA,
            pltpu.SemaphoreType.DMA,
            pltpu.SemaphoreType.DMA,
            pltpu.SemaphoreType.DMA,
        ],
    )
    def sc_embed(ids_hbm, sids_hbm, tok_hbm, seg_hbm, pos_hbm, out_hbm,
                 idx_v, sidx_v, tok0, tok1, pos0, pos1, seg_v,
                 st0, st1, sp0, sp1, so0, so1):
        wid = lax.axis_index("s") * _NC + lax.axis_index("c")
        base = wid * _PER_W
        s0 = base % _S

        toks = (tok0, tok1)
        poss = (pos0, pos1)
        sts = (st0, st1)
        sps = (sp0, sp1)
        sos = (so0, so1)

        pltpu.sync_copy(ids_hbm.at[wid], idx_v)
        pltpu.sync_copy(sids_hbm.at[wid], sidx_v)
        pltpu.sync_copy(seg_hbm, seg_v)

        def gathers(j, b):
            return (
                pltpu.make_async_copy(
                    tok_hbm.at[idx_v.at[j]], toks[b], sts[b]),
                pltpu.make_async_copy(
                    pos_hbm.at[pl.ds(s0 + j * _C, _C)], poss[b], sps[b]),
            )

        def out_copy(j, b):
            return pltpu.make_async_copy(
                toks[b], out_hbm.at[pl.ds(base + j * _C, _C)], sos[b])

        for cp in gathers(0, 0):
            cp.start()

        def outer(i, carry):
            for b in (0, 1):
                j = i * 2 + b
                nb = 1 - b
                for cp in gathers(j, b):
                    cp.wait()

                @pl.when(j >= 1)
                def _wait_prev_out():
                    out_copy(j - 1, nb).wait()

                @pl.when(j + 1 < _NCH)
                def _issue_next():
                    for cp in gathers(j + 1, nb):
                        cp.start()

                tok_b, pos_b = toks[b], poss[b]

                svec = sidx_v[j, :].astype(jnp.float32)
                sidf = [
                    jnp.broadcast_to(svec[r], (_L,)) for r in range(_C)
                ]

                def col_body(cb, carry2):
                    sl = pl.ds(cb * _L, _L)
                    s0v = seg_v[0, sl]
                    dsv = seg_v[1, sl] - s0v
                    for r in range(_C):
                        tok_b[r, sl] = (tok_b[r, sl] + pos_b[r, sl]
                                        + (s0v + sidf[r] * dsv))
                    return carry2

                lax.fori_loop(0, _CVECS, col_body, None)
                out_copy(j, b).start()
            return carry

        lax.fori_loop(0, _NCH // 2, outer, None)
        out_copy(_NCH - 1, (_NCH - 1) % 2).wait()

    return sc_embed


_sc_embed = _make_sc_embed()


@jax.jit
def kernel(input_ids, segment_ids, token_table, segment_table,
           position_table):
    ids = input_ids.reshape(_NW, _NCH, _C).astype(jnp.int32)
    sids = segment_ids.reshape(_NW, _NCH, _C).astype(jnp.int32)
    out = _sc_embed(ids, sids, token_table, segment_table, position_table)
    return out.reshape(_B, _S, _D)
